# baseline (speedup 1.0000x reference)
import functools

import jax
import jax.numpy as jnp
import numpy as np
from jax import lax
from jax.experimental import pallas as pl
from jax.experimental.pallas import tpu as pltpu
from jax.experimental.pallas import tpu_sc as plsc

ROWS = 1024
COLS = 100000
OUT_GROUPS = 1024
W = 98
SG = 784
NA = 64 * SG
NB = COLS - NA
N_WORKERS = 32
ROWS_PER_W = ROWS // N_WORKERS


def _emit_supergroup(buf, base, n_vregs, n_groups, lane):
    acc_s = [jnp.zeros((16,), jnp.float32) for _ in range(n_groups)]
    acc_q = [jnp.zeros((16,), jnp.float32) for _ in range(n_groups)]
    for v in range(n_vregs):
        e0 = 16 * v
        g_lo = e0 // W
        g_hi = (e0 + 15) // W
        xv = buf[pl.ds(base + e0, 16)]
        q = xv * xv
        if g_lo == g_hi:
            acc_s[g_lo] = acc_s[g_lo] + xv
            acc_q[g_lo] = acc_q[g_lo] + q
        else:
            cut = W * g_hi - e0
            in_lo = lane < cut
            x_lo = jnp.where(in_lo, xv, 0.0)
            q_lo = jnp.where(in_lo, q, 0.0)
            acc_s[g_lo] = acc_s[g_lo] + x_lo
            acc_s[g_hi] = acc_s[g_hi] + (xv - x_lo)
            acc_q[g_lo] = acc_q[g_lo] + q_lo
            acc_q[g_hi] = acc_q[g_hi] + (q - q_lo)
    return acc_s, acc_q


def _sc_body(x_hbm, gs_hbm, gsq_hbm, buf_a, buf_b, outs, outq, t_s, t_q,
             sem_a, sem_b):
    cid = lax.axis_index("c")
    sid = lax.axis_index("s")
    wid = sid * 2 + cid
    row0 = wid * ROWS_PER_W
    lane = lax.iota(jnp.int32, 16)
    zero = jnp.zeros((16,), jnp.float32)
    idx_st = [lane + (17 * s) for s in range(16)]
    idx_ld = [lane * 17 + l for l in range(16)]

    def stage(buf, base, n_vregs, n_groups, slot0):
        acc_s, acc_q = _emit_supergroup(buf, base, n_vregs, n_groups, lane)
        for g in range(n_groups):
            plsc.store_scatter(t_s, [idx_st[slot0 + g]], acc_s[g])
            plsc.store_scatter(t_q, [idx_st[slot0 + g]], acc_q[g])

    def flush16(out_base):
        res_s = plsc.load_gather(t_s, [idx_ld[0]])
        res_q = plsc.load_gather(t_q, [idx_ld[0]])
        for l in range(1, 16):
            res_s = res_s + plsc.load_gather(t_s, [idx_ld[l]])
            res_q = res_q + plsc.load_gather(t_q, [idx_ld[l]])
        outs[pl.ds(out_base, 16)] = res_s
        outq[pl.ds(out_base, 16)] = res_q

    def row_body(r, carry):
        row = row0 + r
        cp_a = pltpu.async_copy(x_hbm.at[row, pl.ds(0, NA)], buf_a, sem_a)
        cp_b = pltpu.async_copy(x_hbm.at[row, pl.ds(NA, NB)], buf_b, sem_b)
        cp_a.wait()

        def pair_a(p, c):
            stage(buf_a, (2 * p) * SG, 49, 8, 0)
            stage(buf_a, (2 * p + 1) * SG, 49, 8, 8)
            flush16(p * 16)
            return c

        lax.fori_loop(0, 32, pair_a, 0)
        cp_b.wait()

        def pair_b(p, c):
            stage(buf_b, (2 * p) * SG, 49, 8, 0)
            stage(buf_b, (2 * p + 1) * SG, 49, 8, 8)
            flush16(512 + p * 16)
            return c

        lax.fori_loop(0, 31, pair_b, 0)
        stage(buf_b, 62 * SG, 49, 8, 0)
        stage(buf_b, 63 * SG, 27, 5, 8)
        for s in range(13, 16):
            plsc.store_scatter(t_s, [idx_st[s]], zero)
            plsc.store_scatter(t_q, [idx_st[s]], zero)
        flush16(1008)
        pltpu.sync_copy(outs, gs_hbm.at[row])
        pltpu.sync_copy(outq, gsq_hbm.at[row])
        return carry

    lax.fori_loop(0, ROWS_PER_W, row_body, 0)


def _finalize_body(gs_ref, gsq_ref, out_ref):
    gs = gs_ref[...]
    gsq = gsq_ref[...]
    norm = jnp.sum(gsq, axis=1, keepdims=True)
    total = jnp.sum(gs, axis=1, keepdims=True)
    is_amp = jnp.all(jnp.abs(norm - 1.0) <= (1e-6 + 1e-5))
    out_ref[...] = jnp.where(is_amp, gsq, gs / total)


@jax.jit
def kernel(x):
    mesh = plsc.VectorSubcoreMesh(core_axis_name="c", subcore_axis_name="s")
    gs, gsq = pl.kernel(
        _sc_body,
        mesh=mesh,
        compiler_params=pltpu.CompilerParams(needs_layout_passes=False),
        out_type=[
            jax.ShapeDtypeStruct((ROWS, OUT_GROUPS), jnp.float32),
            jax.ShapeDtypeStruct((ROWS, OUT_GROUPS), jnp.float32),
        ],
        scratch_types=[
            pltpu.VMEM((NA,), jnp.float32),
            pltpu.VMEM((NB,), jnp.float32),
            pltpu.VMEM((OUT_GROUPS,), jnp.float32),
            pltpu.VMEM((OUT_GROUPS,), jnp.float32),
            pltpu.VMEM((16 * 17,), jnp.float32),
            pltpu.VMEM((16 * 17,), jnp.float32),
            pltpu.SemaphoreType.DMA,
            pltpu.SemaphoreType.DMA,
        ],
    )(x)

    out = pl.pallas_call(
        _finalize_body,
        out_shape=jax.ShapeDtypeStruct((ROWS, OUT_GROUPS), jnp.float32),
    )(gs, gsq)
    return out


# speedup vs baseline: 1.5399x; 1.5399x over previous
import functools

import jax
import jax.numpy as jnp
from jax.experimental import pallas as pl
from jax.experimental.pallas import tpu as pltpu

OUT_GROUPS = 1024
GROUPS_PER_BLK = 128


def _group_sums_body(n_cols, cb, x_ref, s_ref, gs_ref, gsq_ref, np_ref):
    j = pl.program_id(1)
    xb = x_ref[...]
    col0 = j * cb
    cols = jax.lax.broadcasted_iota(jnp.int32, xb.shape, 1) + col0
    xb = jnp.where(cols < n_cols, xb, 0.0)
    s = s_ref[...]
    xsq = xb * xb
    xb16 = xb.astype(jnp.bfloat16)
    xsq16 = xsq.astype(jnp.bfloat16)
    dn = (((1,), (0,)), ((), ()))
    gs_ref[...] = jax.lax.dot_general(xb16, s, dn,
                                      preferred_element_type=jnp.float32)
    gsq_ref[...] = jax.lax.dot_general(xsq16, s, dn,
                                       preferred_element_type=jnp.float32)
    part = jnp.broadcast_to(jnp.sum(xsq, axis=1, keepdims=True),
                            np_ref.shape)

    @pl.when(j == 0)
    def _():
        np_ref[...] = part

    @pl.when(j != 0)
    def _():
        np_ref[...] += part


def _finalize_body(gs_ref, gsq_ref, np_ref, out_ref):
    gs = gs_ref[...]
    gsq = gsq_ref[...]
    norm = np_ref[:, :1]
    total = jnp.sum(gs, axis=1, keepdims=True)
    is_amp = jnp.all(jnp.abs(norm - 1.0) <= (1e-6 + 1e-5))
    out_ref[...] = jnp.where(is_amp, gsq, gs / total)


@jax.jit
def kernel(x):
    rows, n_cols = x.shape
    w = -(-n_cols // OUT_GROUPS)
    cb = w * GROUPS_PER_BLK
    nj = -(-OUT_GROUPS // GROUPS_PER_BLK)
    rb = min(512, rows)

    a = jax.lax.broadcasted_iota(jnp.int32, (cb, GROUPS_PER_BLK), 0)
    g = jax.lax.broadcasted_iota(jnp.int32, (cb, GROUPS_PER_BLK), 1)
    sel = ((a >= g * w) & (a < (g + 1) * w)).astype(jnp.bfloat16)

    gs, gsq, nparts = pl.pallas_call(
        functools.partial(_group_sums_body, n_cols, cb),
        grid=(rows // rb, nj),
        in_specs=[
            pl.BlockSpec((rb, cb), lambda i, j: (i, j)),
            pl.BlockSpec((cb, GROUPS_PER_BLK), lambda i, j: (0, 0)),
        ],
        out_specs=[
            pl.BlockSpec((rb, GROUPS_PER_BLK), lambda i, j: (i, j)),
            pl.BlockSpec((rb, GROUPS_PER_BLK), lambda i, j: (i, j)),
            pl.BlockSpec((rb, 128), lambda i, j: (i, 0)),
        ],
        out_shape=[
            jax.ShapeDtypeStruct((rows, OUT_GROUPS), jnp.float32),
            jax.ShapeDtypeStruct((rows, OUT_GROUPS), jnp.float32),
            jax.ShapeDtypeStruct((rows, 128), jnp.float32),
        ],
    )(x, sel)

    out = pl.pallas_call(
        _finalize_body,
        out_shape=jax.ShapeDtypeStruct((rows, OUT_GROUPS), jnp.float32),
    )(gs, gsq, nparts)
    return out
